# baseline (speedup 1.0000x reference)
import jax
import jax.numpy as jnp
from jax.experimental import pallas as pl
from jax.experimental.pallas import tpu as pltpu

_EPS = 1e-6
_LN2 = 0.6931471805599453
_N_TERMS = 6_000_000
_B = 30
_N = 100000
_NBUF = 8


def _slice_sum(pbuf, tbuf, slot):
    loc = pbuf[slot, 0:2, :]
    sc = pbuf[slot, 2:4, :]
    t = tbuf[slot]
    q = sc + (jnp.maximum(sc, _EPS) - sc)
    contrib = jnp.log(q) + jnp.abs(t - loc) / q
    return jnp.sum(contrib)


def _nll_body(pt_hbm, tt_hbm, o_ref, pbuf, tbuf, psem, tsem):
    def start(b, slot):
        pltpu.make_async_copy(pt_hbm.at[b], pbuf.at[slot],
                              psem.at[slot]).start()
        pltpu.make_async_copy(tt_hbm.at[b], tbuf.at[slot],
                              tsem.at[slot]).start()

    for slot in range(_NBUF):
        start(slot, slot)

    def step(b, acc):
        slot = jax.lax.rem(b, _NBUF)
        pltpu.make_async_copy(pt_hbm.at[0], pbuf.at[slot],
                              psem.at[slot]).wait()
        pltpu.make_async_copy(tt_hbm.at[0], tbuf.at[slot],
                              tsem.at[slot]).wait()
        acc = acc + _slice_sum(pbuf, tbuf, slot)
        nb = b + _NBUF

        @pl.when(nb < _B)
        def _():
            start(nb, slot)

        return acc

    acc = jax.lax.fori_loop(0, _B, step, jnp.float32(0.0))
    o_ref[0, 0] = acc * (1.0 / _N_TERMS) + _LN2


def kernel(pred, target):
    pt = jnp.swapaxes(pred, 1, 2)
    tt = jnp.swapaxes(target, 1, 2)
    out = pl.pallas_call(
        _nll_body,
        in_specs=[
            pl.BlockSpec(memory_space=pltpu.MemorySpace.HBM),
            pl.BlockSpec(memory_space=pltpu.MemorySpace.HBM),
        ],
        out_specs=pl.BlockSpec(memory_space=pltpu.SMEM),
        out_shape=jax.ShapeDtypeStruct((1, 1), jnp.float32),
        scratch_shapes=[
            pltpu.VMEM((_NBUF, 4, _N), jnp.float32),
            pltpu.VMEM((_NBUF, 2, _N), jnp.float32),
            pltpu.SemaphoreType.DMA((_NBUF,)),
            pltpu.SemaphoreType.DMA((_NBUF,)),
        ],
    )(pt, tt)
    return out[0, 0]


# speedup vs baseline: 1.0095x; 1.0095x over previous
import jax
import jax.numpy as jnp
from jax.experimental import pallas as pl
from jax.experimental.pallas import tpu as pltpu

_EPS = 1e-6
_LN2 = 0.6931471805599453
_N_TERMS = 6_000_000
_B = 30
_N = 100000
_K = 3
_S = _B // _K


def _nll_body(*refs):
    o_ref = refs[-1]
    i = pl.program_id(0)
    s = jnp.float32(0.0)
    _CH = 12800
    bounds = list(range(0, _N, _CH)) + [_N]
    for k in range(_K):
        p_ref = refs[2 * k]
        t_ref = refs[2 * k + 1]
        for c0, c1 in zip(bounds[:-1], bounds[1:]):
            loc = p_ref[0, 0:2, c0:c1]
            sc = p_ref[0, 2:4, c0:c1]
            t = t_ref[0, :, c0:c1]
            q = sc + (jnp.maximum(sc, _EPS) - sc)
            contrib = jnp.log(q) + jnp.abs(t - loc) / q
            s = s + jnp.sum(contrib)
    tot = jnp.where(i == 0, 0.0, o_ref[0, 0]) + s
    o_ref[0, 0] = jnp.where(i == _S - 1,
                            tot * (1.0 / _N_TERMS) + _LN2, tot)


def kernel(pred, target):
    pt = jnp.swapaxes(pred, 1, 2)
    tt = jnp.swapaxes(target, 1, 2)
    in_specs = []
    ops = []
    for k in range(_K):
        in_specs.append(
            pl.BlockSpec((1, 4, _N), lambda i, k=k: (k * _S + i, 0, 0)))
        in_specs.append(
            pl.BlockSpec((1, 2, _N), lambda i, k=k: (k * _S + i, 0, 0)))
        ops.extend([pt, tt])
    out = pl.pallas_call(
        _nll_body,
        grid=(_S,),
        in_specs=in_specs,
        out_specs=pl.BlockSpec(memory_space=pltpu.SMEM),
        out_shape=jax.ShapeDtypeStruct((1, 1), jnp.float32),
        compiler_params=pltpu.CompilerParams(
            dimension_semantics=("arbitrary",)),
    )(*ops)
    return out[0, 0]
